# baseline (speedup 1.0000x reference)
import functools

import jax
import jax.numpy as jnp
from jax import lax
from jax.experimental import pallas as pl
from jax.experimental.pallas import tpu as pltpu
from jax.experimental.pallas import tpu_sc as plsc

NC = 2
NS = 16
NW = NC * NS
IB = 128
IXR = 8


def _mesh():
    return plsc.VectorSubcoreMesh(
        core_axis_name="c", subcore_axis_name="s",
        num_cores=NC, num_subcores=NS)


@functools.lru_cache(maxsize=None)
def _make_agg(D, EW, NP):
    NO = EW // (IB * IXR)
    ZR = NP // NS

    def body(y_hbm, srcr_hbm, dstr_hbm, zero_hbm, out_hbm,
             src_v, dst_v, rows0, rows1, acc_sh, sem0, sem1):
        c = lax.axis_index("c")
        s = lax.axis_index("s")
        wid = s * NC + c
        pltpu.sync_copy(zero_hbm, acc_sh.at[pl.ds(s * ZR, ZR)])
        plsc.subcore_barrier()

        row_base = wid * (EW // IB)
        bufs = (rows0, rows1)
        sems = (sem0, sem1)

        @pl.loop(0, NO)
        def _blk(o):
            r0 = row_base + o * IXR
            pltpu.sync_copy(srcr_hbm.at[pl.ds(r0, IXR)], src_v)
            pltpu.sync_copy(dstr_hbm.at[pl.ds(r0, IXR)], dst_v)
            cps = [None] * IXR
            cps[0] = pltpu.async_copy(y_hbm.at[src_v.at[0]], bufs[0], sem0)
            for j in range(IXR):
                if j + 1 < IXR:
                    k = (j + 1) % 2
                    cps[j + 1] = pltpu.async_copy(
                        y_hbm.at[src_v.at[j + 1]], bufs[k], sems[k])
                cps[j].wait()
                pltpu.sync_copy(bufs[j % 2], acc_sh.at[dst_v.at[j]],
                                add=True)

        plsc.subcore_barrier()
        pltpu.sync_copy(acc_sh.at[pl.ds(s * ZR, ZR)],
                        out_hbm.at[c, pl.ds(s * ZR, ZR)])

    return pl.kernel(
        body,
        out_type=jax.ShapeDtypeStruct((NC, NP, D), jnp.float32),
        mesh=_mesh(),
        scratch_types=[
            pltpu.VMEM((IXR, IB), jnp.int32),
            pltpu.VMEM((IXR, IB), jnp.int32),
            pltpu.VMEM((IB, D), jnp.float32),
            pltpu.VMEM((IB, D), jnp.float32),
            pltpu.VMEM_SHARED((NP, D), jnp.float32),
            pltpu.SemaphoreType.DMA,
            pltpu.SemaphoreType.DMA,
        ],
    )


@functools.lru_cache(maxsize=None)
def _make_count(D, EW, NP):
    CH = 256
    SUB = CH // IB
    NCH = EW // CH
    ZR = NP // NS

    def body(dstr_hbm, ones_hbm, zero_hbm, out_hbm, dst_v, ones_v, cnt_sh):
        c = lax.axis_index("c")
        s = lax.axis_index("s")
        wid = s * NC + c
        pltpu.sync_copy(zero_hbm, cnt_sh.at[pl.ds(s * ZR, ZR)])
        pltpu.sync_copy(ones_hbm, ones_v)
        plsc.subcore_barrier()

        row_base = wid * (EW // IB)

        @pl.loop(0, NCH)
        def _chunk(i):
            r0 = row_base + i * SUB
            pltpu.sync_copy(dstr_hbm.at[pl.ds(r0, SUB)], dst_v)
            for j in range(SUB):
                pltpu.sync_copy(ones_v, cnt_sh.at[dst_v.at[j]], add=True)

        plsc.subcore_barrier()
        pltpu.sync_copy(cnt_sh.at[pl.ds(s * ZR, ZR)],
                        out_hbm.at[c, pl.ds(s * ZR, ZR)])

    return pl.kernel(
        body,
        out_type=jax.ShapeDtypeStruct((NC, NP, D), jnp.float32),
        mesh=_mesh(),
        scratch_types=[
            pltpu.VMEM((SUB, IB), jnp.int32),
            pltpu.VMEM((IB, D), jnp.float32),
            pltpu.VMEM_SHARED((NP, D), jnp.float32),
        ],
    )


BR = 2000


def _tc_pre(x, Wl, Wr, b):
    N_, D = x.shape

    def body(x_ref, wl_ref, wr_ref, b_ref, yl_ref, yr_ref):
        xb = x_ref[...]
        yl_ref[...] = jnp.dot(xb, wl_ref[...],
                              preferred_element_type=jnp.float32)
        yr_ref[...] = jnp.dot(xb, wr_ref[...],
                              preferred_element_type=jnp.float32) + b_ref[...]

    return pl.pallas_call(
        body,
        grid=(N_ // BR,),
        in_specs=[
            pl.BlockSpec((BR, D), lambda i: (i, 0)),
            pl.BlockSpec((D, D), lambda i: (0, 0)),
            pl.BlockSpec((D, D), lambda i: (0, 0)),
            pl.BlockSpec((1, D), lambda i: (0, 0)),
        ],
        out_specs=[pl.BlockSpec((BR, D), lambda i: (i, 0))] * 2,
        out_shape=[jax.ShapeDtypeStruct((N_, D), jnp.float32)] * 2,
    )(x, Wl, Wr, b.reshape(1, D))


def _tc_mid(a0, a1, cnt0, cnt1, yrp, Wl, Wr, b):
    N_, D = a0.shape

    def body(a0_ref, a1_ref, c0_ref, c1_ref, yrp_ref, wl_ref, wr_ref, b_ref,
             yl_ref, yr_ref):
        cnt = c0_ref[...] + c1_ref[...]
        inv = 1.0 / jnp.maximum(cnt[:, 0:1], 1.0)
        h = jnp.maximum((a0_ref[...] + a1_ref[...]) * inv + yrp_ref[...], 0.0)
        yl_ref[...] = jnp.dot(h, wl_ref[...],
                              preferred_element_type=jnp.float32)
        yr_ref[...] = jnp.dot(h, wr_ref[...],
                              preferred_element_type=jnp.float32) + b_ref[...]

    return pl.pallas_call(
        body,
        grid=(N_ // BR,),
        in_specs=[
            pl.BlockSpec((BR, D), lambda i: (i, 0)),
            pl.BlockSpec((BR, D), lambda i: (i, 0)),
            pl.BlockSpec((BR, D), lambda i: (i, 0)),
            pl.BlockSpec((BR, D), lambda i: (i, 0)),
            pl.BlockSpec((BR, D), lambda i: (i, 0)),
            pl.BlockSpec((D, D), lambda i: (0, 0)),
            pl.BlockSpec((D, D), lambda i: (0, 0)),
            pl.BlockSpec((1, D), lambda i: (0, 0)),
        ],
        out_specs=[pl.BlockSpec((BR, D), lambda i: (i, 0))] * 2,
        out_shape=[jax.ShapeDtypeStruct((N_, D), jnp.float32)] * 2,
    )(a0, a1, cnt0, cnt1, yrp, Wl, Wr, b.reshape(1, D))


def _tc_post(a0, a1, cnt0, cnt1, yrp):
    N_, D = a0.shape

    def body(a0_ref, a1_ref, c0_ref, c1_ref, yrp_ref, out_ref):
        cnt = c0_ref[...] + c1_ref[...]
        inv = 1.0 / jnp.maximum(cnt[:, 0:1], 1.0)
        out_ref[...] = (a0_ref[...] + a1_ref[...]) * inv + yrp_ref[...]

    return pl.pallas_call(
        body,
        grid=(N_ // BR,),
        in_specs=[
            pl.BlockSpec((BR, D), lambda i: (i, 0)),
            pl.BlockSpec((BR, D), lambda i: (i, 0)),
            pl.BlockSpec((BR, D), lambda i: (i, 0)),
            pl.BlockSpec((BR, D), lambda i: (i, 0)),
            pl.BlockSpec((BR, D), lambda i: (i, 0)),
        ],
        out_specs=pl.BlockSpec((BR, D), lambda i: (i, 0)),
        out_shape=jax.ShapeDtypeStruct((N_, D), jnp.float32),
    )(a0, a1, cnt0, cnt1, yrp)


def kernel(x, edge_index, Wl0, Wr0, b0, Wl1, Wr1, b1, Wl2, Wr2, b2):
    N_, D = x.shape
    E_ = edge_index.shape[1]
    src = edge_index[0]
    dst = edge_index[1]

    EW = -(-E_ // (NW * IB * IXR)) * (IB * IXR)
    E_pad = EW * NW
    NP = -(-(N_ + 1) // (NS * 8)) * (NS * 8)

    pad = E_pad - E_
    src_p = jnp.concatenate([src, jnp.zeros((pad,), jnp.int32)])
    dst_p = jnp.concatenate([dst, jnp.full((pad,), N_, jnp.int32)])
    srcr = src_p.reshape(E_pad // IB, IB)
    dstr = dst_p.reshape(E_pad // IB, IB)
    ZR = NP // NS
    zeroD = jnp.zeros((ZR, D), jnp.float32)
    onesD = jnp.ones((IB, D), jnp.float32)

    agg = _make_agg(D, EW, NP)
    cntp = _make_count(D, EW, NP)(dstr, onesD, zeroD)
    cnt0 = cntp[0, :N_]
    cnt1 = cntp[1, :N_]

    yl, yr = _tc_pre(x, Wl0, Wr0, b0)
    s = agg(yl, srcr, dstr, zeroD)
    yl, yr = _tc_mid(s[0, :N_], s[1, :N_], cnt0, cnt1, yr, Wl1, Wr1, b1)
    s = agg(yl, srcr, dstr, zeroD)
    yl, yr = _tc_mid(s[0, :N_], s[1, :N_], cnt0, cnt1, yr, Wl2, Wr2, b2)
    s = agg(yl, srcr, dstr, zeroD)
    return _tc_post(s[0, :N_], s[1, :N_], cnt0, cnt1, yr)


# speedup vs baseline: 1.0196x; 1.0196x over previous
import functools

import jax
import jax.numpy as jnp
from jax import lax
from jax.experimental import pallas as pl
from jax.experimental.pallas import tpu as pltpu
from jax.experimental.pallas import tpu_sc as plsc

NC = 2
NS = 16
NW = NC * NS
IB = 128
IXR = 16


def _mesh():
    return plsc.VectorSubcoreMesh(
        core_axis_name="c", subcore_axis_name="s",
        num_cores=NC, num_subcores=NS)


@functools.lru_cache(maxsize=None)
def _make_agg(D, EW, NP):
    NO = EW // (IB * IXR)
    ZR = NP // NS

    def body(y_hbm, srcr_hbm, dstr_hbm, zero_hbm, out_hbm,
             src_v, dst_v, rows0, rows1, acc_sh, sem0, sem1):
        c = lax.axis_index("c")
        s = lax.axis_index("s")
        wid = s * NC + c
        pltpu.sync_copy(zero_hbm, acc_sh.at[pl.ds(s * ZR, ZR)])
        plsc.subcore_barrier()

        row_base = wid * (EW // IB)
        bufs = (rows0, rows1)
        sems = (sem0, sem1)

        @pl.loop(0, NO)
        def _blk(o):
            r0 = row_base + o * IXR
            pltpu.sync_copy(srcr_hbm.at[pl.ds(r0, IXR)], src_v)
            pltpu.sync_copy(dstr_hbm.at[pl.ds(r0, IXR)], dst_v)
            cps = [None] * IXR
            cps[0] = pltpu.async_copy(y_hbm.at[src_v.at[0]], bufs[0], sem0)
            for j in range(IXR):
                if j + 1 < IXR:
                    k = (j + 1) % 2
                    cps[j + 1] = pltpu.async_copy(
                        y_hbm.at[src_v.at[j + 1]], bufs[k], sems[k])
                cps[j].wait()
                pltpu.sync_copy(bufs[j % 2], acc_sh.at[dst_v.at[j]],
                                add=True)

        plsc.subcore_barrier()
        pltpu.sync_copy(acc_sh.at[pl.ds(s * ZR, ZR)],
                        out_hbm.at[c, pl.ds(s * ZR, ZR)])

    return pl.kernel(
        body,
        out_type=jax.ShapeDtypeStruct((NC, NP, D), jnp.float32),
        mesh=_mesh(),
        scratch_types=[
            pltpu.VMEM((IXR, IB), jnp.int32),
            pltpu.VMEM((IXR, IB), jnp.int32),
            pltpu.VMEM((IB, D), jnp.float32),
            pltpu.VMEM((IB, D), jnp.float32),
            pltpu.VMEM_SHARED((NP, D), jnp.float32),
            pltpu.SemaphoreType.DMA,
            pltpu.SemaphoreType.DMA,
        ],
    )


@functools.lru_cache(maxsize=None)
def _make_count(D, EW, NP):
    CH = 256
    SUB = CH // IB
    NCH = EW // CH
    ZR = NP // NS

    def body(dstr_hbm, ones_hbm, zero_hbm, out_hbm, dst_v, ones_v, cnt_sh):
        c = lax.axis_index("c")
        s = lax.axis_index("s")
        wid = s * NC + c
        pltpu.sync_copy(zero_hbm, cnt_sh.at[pl.ds(s * ZR, ZR)])
        pltpu.sync_copy(ones_hbm, ones_v)
        plsc.subcore_barrier()

        row_base = wid * (EW // IB)

        @pl.loop(0, NCH)
        def _chunk(i):
            r0 = row_base + i * SUB
            pltpu.sync_copy(dstr_hbm.at[pl.ds(r0, SUB)], dst_v)
            for j in range(SUB):
                pltpu.sync_copy(ones_v, cnt_sh.at[dst_v.at[j]], add=True)

        plsc.subcore_barrier()
        pltpu.sync_copy(cnt_sh.at[pl.ds(s * ZR, ZR)],
                        out_hbm.at[c, pl.ds(s * ZR, ZR)])

    return pl.kernel(
        body,
        out_type=jax.ShapeDtypeStruct((NC, NP, D), jnp.float32),
        mesh=_mesh(),
        scratch_types=[
            pltpu.VMEM((SUB, IB), jnp.int32),
            pltpu.VMEM((IB, D), jnp.float32),
            pltpu.VMEM_SHARED((NP, D), jnp.float32),
        ],
    )


BR = 2000


def _tc_pre(x, Wl, Wr, b):
    N_, D = x.shape

    def body(x_ref, wl_ref, wr_ref, b_ref, yl_ref, yr_ref):
        xb = x_ref[...]
        yl_ref[...] = jnp.dot(xb, wl_ref[...],
                              preferred_element_type=jnp.float32)
        yr_ref[...] = jnp.dot(xb, wr_ref[...],
                              preferred_element_type=jnp.float32) + b_ref[...]

    return pl.pallas_call(
        body,
        grid=(N_ // BR,),
        in_specs=[
            pl.BlockSpec((BR, D), lambda i: (i, 0)),
            pl.BlockSpec((D, D), lambda i: (0, 0)),
            pl.BlockSpec((D, D), lambda i: (0, 0)),
            pl.BlockSpec((1, D), lambda i: (0, 0)),
        ],
        out_specs=[pl.BlockSpec((BR, D), lambda i: (i, 0))] * 2,
        out_shape=[jax.ShapeDtypeStruct((N_, D), jnp.float32)] * 2,
    )(x, Wl, Wr, b.reshape(1, D))


def _tc_mid(a0, a1, cnt0, cnt1, yrp, Wl, Wr, b):
    N_, D = a0.shape

    def body(a0_ref, a1_ref, c0_ref, c1_ref, yrp_ref, wl_ref, wr_ref, b_ref,
             yl_ref, yr_ref):
        cnt = c0_ref[...] + c1_ref[...]
        inv = 1.0 / jnp.maximum(cnt[:, 0:1], 1.0)
        h = jnp.maximum((a0_ref[...] + a1_ref[...]) * inv + yrp_ref[...], 0.0)
        yl_ref[...] = jnp.dot(h, wl_ref[...],
                              preferred_element_type=jnp.float32)
        yr_ref[...] = jnp.dot(h, wr_ref[...],
                              preferred_element_type=jnp.float32) + b_ref[...]

    return pl.pallas_call(
        body,
        grid=(N_ // BR,),
        in_specs=[
            pl.BlockSpec((BR, D), lambda i: (i, 0)),
            pl.BlockSpec((BR, D), lambda i: (i, 0)),
            pl.BlockSpec((BR, D), lambda i: (i, 0)),
            pl.BlockSpec((BR, D), lambda i: (i, 0)),
            pl.BlockSpec((BR, D), lambda i: (i, 0)),
            pl.BlockSpec((D, D), lambda i: (0, 0)),
            pl.BlockSpec((D, D), lambda i: (0, 0)),
            pl.BlockSpec((1, D), lambda i: (0, 0)),
        ],
        out_specs=[pl.BlockSpec((BR, D), lambda i: (i, 0))] * 2,
        out_shape=[jax.ShapeDtypeStruct((N_, D), jnp.float32)] * 2,
    )(a0, a1, cnt0, cnt1, yrp, Wl, Wr, b.reshape(1, D))


def _tc_post(a0, a1, cnt0, cnt1, yrp):
    N_, D = a0.shape

    def body(a0_ref, a1_ref, c0_ref, c1_ref, yrp_ref, out_ref):
        cnt = c0_ref[...] + c1_ref[...]
        inv = 1.0 / jnp.maximum(cnt[:, 0:1], 1.0)
        out_ref[...] = (a0_ref[...] + a1_ref[...]) * inv + yrp_ref[...]

    return pl.pallas_call(
        body,
        grid=(N_ // BR,),
        in_specs=[
            pl.BlockSpec((BR, D), lambda i: (i, 0)),
            pl.BlockSpec((BR, D), lambda i: (i, 0)),
            pl.BlockSpec((BR, D), lambda i: (i, 0)),
            pl.BlockSpec((BR, D), lambda i: (i, 0)),
            pl.BlockSpec((BR, D), lambda i: (i, 0)),
        ],
        out_specs=pl.BlockSpec((BR, D), lambda i: (i, 0)),
        out_shape=jax.ShapeDtypeStruct((N_, D), jnp.float32),
    )(a0, a1, cnt0, cnt1, yrp)


def kernel(x, edge_index, Wl0, Wr0, b0, Wl1, Wr1, b1, Wl2, Wr2, b2):
    N_, D = x.shape
    E_ = edge_index.shape[1]
    src = edge_index[0]
    dst = edge_index[1]

    EW = -(-E_ // (NW * IB * IXR)) * (IB * IXR)
    E_pad = EW * NW
    NP = -(-(N_ + 1) // (NS * 8)) * (NS * 8)

    pad = E_pad - E_
    src_p = jnp.concatenate([src, jnp.zeros((pad,), jnp.int32)])
    dst_p = jnp.concatenate([dst, jnp.full((pad,), N_, jnp.int32)])
    srcr = src_p.reshape(E_pad // IB, IB)
    dstr = dst_p.reshape(E_pad // IB, IB)
    ZR = NP // NS
    zeroD = jnp.zeros((ZR, D), jnp.float32)
    onesD = jnp.ones((IB, D), jnp.float32)

    agg = _make_agg(D, EW, NP)
    cntp = _make_count(D, EW, NP)(dstr, onesD, zeroD)
    cnt0 = cntp[0, :N_]
    cnt1 = cntp[1, :N_]

    yl, yr = _tc_pre(x, Wl0, Wr0, b0)
    s = agg(yl, srcr, dstr, zeroD)
    yl, yr = _tc_mid(s[0, :N_], s[1, :N_], cnt0, cnt1, yr, Wl1, Wr1, b1)
    s = agg(yl, srcr, dstr, zeroD)
    yl, yr = _tc_mid(s[0, :N_], s[1, :N_], cnt0, cnt1, yr, Wl2, Wr2, b2)
    s = agg(yl, srcr, dstr, zeroD)
    return _tc_post(s[0, :N_], s[1, :N_], cnt0, cnt1, yr)
